# baseline (speedup 1.0000x reference)
import jax
import jax.numpy as jnp
from jax.experimental import pallas as pl
from jax.experimental.pallas import tpu as pltpu

_B, _N, _C, _H, _W = 2, 512, 256, 7, 7
_D_IN = _C * _H * _W
_D_HID = 1024
_OUT = 16
_M = _B * _N

_CB = 8
_TK = _CB * _H * _W


def _mlp_kernel(x_ref, w1_ref, b1_ref, w2_ref, b2_ref, wh_ref, bh_ref,
                o_ref, acc_ref):
    c = pl.program_id(1)

    @pl.when(c == 0)
    def _init():
        acc_ref[...] = jnp.zeros_like(acc_ref)

    x2d = x_ref[...].reshape(_N, _TK)
    acc_ref[...] += jnp.dot(x2d, w1_ref[...],
                            preferred_element_type=jnp.float32)

    @pl.when(c == pl.num_programs(1) - 1)
    def _finish():
        h1 = jnp.maximum(acc_ref[...] + b1_ref[...], 0.0)
        h2 = jnp.maximum(
            jnp.dot(h1, w2_ref[...], preferred_element_type=jnp.float32)
            + b2_ref[...], 0.0)
        res = (jnp.dot(h2, wh_ref[...], preferred_element_type=jnp.float32)
               + bh_ref[...])
        o_ref[...] = res[None, :, :]


def kernel(aligned_feat, W1, b1, W2, b2, Wc, bc, Wr, br):
    Wh = jnp.concatenate([Wc, Wr], axis=1)
    bh = jnp.concatenate([bc, br]).reshape(1, _OUT)
    b1r = b1.reshape(1, _D_HID)
    b2r = b2.reshape(1, _D_HID)

    grid = (_B, _C // _CB)
    out = pl.pallas_call(
        _mlp_kernel,
        grid=grid,
        in_specs=[
            pl.BlockSpec((1, _N, _CB, _H, _W), lambda b, c: (b, 0, c, 0, 0)),
            pl.BlockSpec((_TK, _D_HID), lambda b, c: (c, 0)),
            pl.BlockSpec((1, _D_HID), lambda b, c: (0, 0)),
            pl.BlockSpec((_D_HID, _D_HID), lambda b, c: (0, 0)),
            pl.BlockSpec((1, _D_HID), lambda b, c: (0, 0)),
            pl.BlockSpec((_D_HID, _OUT), lambda b, c: (0, 0)),
            pl.BlockSpec((1, _OUT), lambda b, c: (0, 0)),
        ],
        out_specs=pl.BlockSpec((1, _N, _OUT), lambda b, c: (b, 0, 0)),
        out_shape=jax.ShapeDtypeStruct((_B, _N, _OUT), jnp.float32),
        scratch_shapes=[pltpu.VMEM((_N, _D_HID), jnp.float32)],
        compiler_params=pltpu.CompilerParams(
            dimension_semantics=("parallel", "arbitrary")),
    )(aligned_feat, W1, b1r, W2, b2r, Wh, bh)
    return out


# speedup vs baseline: 4.8451x; 4.8451x over previous
import jax
import jax.numpy as jnp
from jax.experimental import pallas as pl
from jax.experimental.pallas import tpu as pltpu

_B, _N, _C, _H, _W = 2, 512, 256, 7, 7
_D_IN = _C * _H * _W
_D_HID = 1024
_OUT = 16
_M = _B * _N

_TM = 1024
_TK = 1792


def _mlp_kernel(x_ref, w1_ref, b1_ref, w2_ref, b2_ref, wh_ref, bh_ref,
                o_ref, acc_ref):
    k = pl.program_id(1)

    @pl.when(k == 0)
    def _init():
        acc_ref[...] = jnp.zeros_like(acc_ref)

    acc_ref[...] += jnp.dot(x_ref[...], w1_ref[...],
                            preferred_element_type=jnp.float32)

    @pl.when(k == pl.num_programs(1) - 1)
    def _finish():
        h1 = jnp.maximum(acc_ref[...] + b1_ref[...], 0.0)
        h2 = jnp.maximum(
            jnp.dot(h1, w2_ref[...], preferred_element_type=jnp.float32)
            + b2_ref[...], 0.0)
        o_ref[...] = (jnp.dot(h2, wh_ref[...],
                              preferred_element_type=jnp.float32)
                      + bh_ref[...])


def kernel(aligned_feat, W1, b1, W2, b2, Wc, bc, Wr, br):
    x = aligned_feat.reshape(_M, _D_IN)
    Wh = jnp.concatenate([Wc, Wr], axis=1)
    bh = jnp.concatenate([bc, br]).reshape(1, _OUT)
    b1r = b1.reshape(1, _D_HID)
    b2r = b2.reshape(1, _D_HID)

    grid = (_M // _TM, _D_IN // _TK)
    out = pl.pallas_call(
        _mlp_kernel,
        grid=grid,
        in_specs=[
            pl.BlockSpec((_TM, _TK), lambda m, k: (m, k)),
            pl.BlockSpec((_TK, _D_HID), lambda m, k: (k, 0)),
            pl.BlockSpec((1, _D_HID), lambda m, k: (0, 0)),
            pl.BlockSpec((_D_HID, _D_HID), lambda m, k: (0, 0)),
            pl.BlockSpec((1, _D_HID), lambda m, k: (0, 0)),
            pl.BlockSpec((_D_HID, _OUT), lambda m, k: (0, 0)),
            pl.BlockSpec((1, _OUT), lambda m, k: (0, 0)),
        ],
        out_specs=pl.BlockSpec((_TM, _OUT), lambda m, k: (m, 0)),
        out_shape=jax.ShapeDtypeStruct((_M, _OUT), jnp.float32),
        scratch_shapes=[pltpu.VMEM((_TM, _D_HID), jnp.float32)],
        compiler_params=pltpu.CompilerParams(
            dimension_semantics=("parallel", "arbitrary"),
            allow_input_fusion=[True, False, False, False, False, False,
                                False]),
    )(x, W1, b1r, W2, b2r, Wh, bh)
    return out.reshape(_B, _N, _OUT)
